# baseline (speedup 1.0000x reference)
import functools

import jax
import jax.numpy as jnp
from jax import lax
from jax.experimental import pallas as pl
from jax.experimental.pallas import tpu as pltpu
from jax.experimental.pallas import tpu_sc as plsc

_B, _N, _G, _K = 8, 8192, 512, 32
_GB = 128
_NW = 32
_RW = (_B * _G) // _NW
_SW = _RW * _K


def _fps_body(x_ref, y_ref, z_ref, cx_ref, cy_ref, cz_ref):
    x = x_ref[0]
    y = y_ref[0]
    z = z_ref[0]
    nb = x.shape[0]
    col = jax.lax.broadcasted_iota(jnp.int32, (nb, _N), 1)
    gcol = jax.lax.broadcasted_iota(jnp.int32, (nb, _G), 1)

    def step(t, carry):
        dist, far, cxs, cys, czs = carry
        sel = col == far
        cx = jnp.sum(jnp.where(sel, x, 0.0), axis=1, keepdims=True)
        cy = jnp.sum(jnp.where(sel, y, 0.0), axis=1, keepdims=True)
        cz = jnp.sum(jnp.where(sel, z, 0.0), axis=1, keepdims=True)
        gsel = gcol == t
        cxs = jnp.where(gsel, cx, cxs)
        cys = jnp.where(gsel, cy, cys)
        czs = jnp.where(gsel, cz, czs)
        dx = x - cx
        dy = y - cy
        dz = z - cz
        d = (dx * dx + dy * dy) + dz * dz
        dist = jnp.minimum(dist, d)
        m = jnp.max(dist, axis=1, keepdims=True)
        far = jnp.min(jnp.where(dist == m, col, _N), axis=1, keepdims=True)
        return dist, far, cxs, cys, czs

    dist0 = jnp.full((nb, _N), 1e10, jnp.float32)
    far0 = jnp.zeros((nb, 1), jnp.int32)
    c0 = jnp.zeros((nb, _G), jnp.float32)
    _, _, cxs, cys, czs = jax.lax.fori_loop(
        0, _G, step, (dist0, far0, c0, c0, c0)
    )
    cx_ref[...] = cxs[None]
    cy_ref[...] = cys[None]
    cz_ref[...] = czs[None]


def _rne_bf16(v):
    u = jax.lax.bitcast_convert_type(v, jnp.uint32)
    r = (u + jnp.uint32(0x7FFF) + ((u >> 16) & jnp.uint32(1))) & jnp.uint32(
        0xFFFF0000
    )
    return jax.lax.bitcast_convert_type(r, jnp.float32)


def _twosum(a, b):
    s = a + b
    bb = s - a
    err = (a - (s - bb)) + (b - bb)
    return s, err


def _knn_body(x_ref, y_ref, z_ref, cx_ref, cy_ref, cz_ref, out_ref):
    x = x_ref[0]
    y = y_ref[0]
    z = z_ref[0]
    cx = cx_ref[0]
    cy = cy_ref[0]
    cz = cz_ref[0]
    p2 = (x * x + y * y) + z * z
    c2 = (cx * cx + cy * cy) + cz * cz
    xb, yb, zb = _rne_bf16(x), _rne_bf16(y), _rne_bf16(z)
    cxb, cyb, czb = _rne_bf16(cx), _rne_bf16(cy), _rne_bf16(cz)
    p0 = cxb * xb
    p1 = cyb * yb
    p2b = czb * zb
    s1, e1 = _twosum(p0, p1)
    s2, e2 = _twosum(s1, p2b)
    dot = s2 + (e1 + e2)
    d2 = (c2 + p2) - 2.0 * dot
    col = jax.lax.broadcasted_iota(jnp.int32, (_GB, _N), 1)
    jcol = jax.lax.broadcasted_iota(jnp.int32, (_GB, _K), 1)
    inds = jnp.zeros((_GB, _K), jnp.int32)
    for j in range(_K):
        m = jnp.min(d2, axis=1, keepdims=True)
        idxc = jnp.where(d2 == m, col, _N)
        idx = jnp.min(idxc, axis=1, keepdims=True)
        d2 = jnp.where(idxc == idx, jnp.float32(jnp.inf), d2)
        inds = jnp.where(jcol == j, idx, inds)
    out_ref[...] = inds[None]


def _fps_centers(x, y, z):
    nb = _B // 2
    full = pl.BlockSpec((1, nb, _N), lambda i: (i, 0, 0))
    outs = jax.ShapeDtypeStruct((2, nb, _G), jnp.float32)
    cx, cy, cz = pl.pallas_call(
        _fps_body,
        grid=(2,),
        in_specs=[full, full, full],
        out_specs=[pl.BlockSpec((1, nb, _G), lambda i: (i, 0, 0))] * 3,
        out_shape=[outs, outs, outs],
        compiler_params=pltpu.CompilerParams(
            dimension_semantics=("parallel",)
        ),
    )(
        x.reshape(2, nb, _N),
        y.reshape(2, nb, _N),
        z.reshape(2, nb, _N),
    )
    return (
        cx.reshape(_B, _G),
        cy.reshape(_B, _G),
        cz.reshape(_B, _G),
    )


def _knn_indices(x, y, z, cx, cy, cz):
    row = pl.BlockSpec((1, 1, _N), lambda b, g: (b, 0, 0))
    cen = pl.BlockSpec((1, _GB, 1), lambda b, g: (b, g, 0))
    return pl.pallas_call(
        _knn_body,
        grid=(_B, _G // _GB),
        in_specs=[row, row, row, cen, cen, cen],
        out_specs=pl.BlockSpec((1, _GB, _K), lambda b, g: (b, g, 0)),
        out_shape=jax.ShapeDtypeStruct((_B, _G, _K), jnp.int32),
        compiler_params=pltpu.CompilerParams(
            dimension_semantics=("parallel", "parallel")
        ),
    )(
        x.reshape(_B, 1, _N),
        y.reshape(_B, 1, _N),
        z.reshape(_B, 1, _N),
        cx.reshape(_B, _G, 1),
        cy.reshape(_B, _G, 1),
        cz.reshape(_B, _G, 1),
    )


def _sc_gather(x, y, z, idx_flat, cxe, cye, cze):
    mesh = plsc.VectorSubcoreMesh(core_axis_name="c", subcore_axis_name="s")
    out_t = jax.ShapeDtypeStruct((_B * _G * _K,), jnp.float32)

    @functools.partial(
        pl.kernel,
        mesh=mesh,
        out_type=[out_t, out_t, out_t],
        compiler_params=pltpu.CompilerParams(needs_layout_passes=False),
        scratch_types=[
            pltpu.VMEM((_SW,), jnp.int32),
            pltpu.VMEM((_N,), jnp.float32),
            pltpu.VMEM((_N,), jnp.float32),
            pltpu.VMEM((_N,), jnp.float32),
            pltpu.VMEM((_SW,), jnp.float32),
            pltpu.VMEM((_SW,), jnp.float32),
            pltpu.VMEM((_SW,), jnp.float32),
            pltpu.VMEM((_SW,), jnp.float32),
            pltpu.VMEM((_SW,), jnp.float32),
            pltpu.VMEM((_SW,), jnp.float32),
        ],
    )
    def k(x_h, y_h, z_h, idx_h, cxe_h, cye_h, cze_h, nx_h, ny_h, nz_h,
          idx_v, x_v, y_v, z_v, cx_v, cy_v, cz_v, nx_v, ny_v, nz_v):
        wid = lax.axis_index("s") * 2 + lax.axis_index("c")
        base = wid * _SW
        b = wid // (_NW // _B)
        pltpu.sync_copy(x_h.at[pl.ds(b * _N, _N)], x_v)
        pltpu.sync_copy(y_h.at[pl.ds(b * _N, _N)], y_v)
        pltpu.sync_copy(z_h.at[pl.ds(b * _N, _N)], z_v)
        pltpu.sync_copy(idx_h.at[pl.ds(base, _SW)], idx_v)
        pltpu.sync_copy(cxe_h.at[pl.ds(base, _SW)], cx_v)
        pltpu.sync_copy(cye_h.at[pl.ds(base, _SW)], cy_v)
        pltpu.sync_copy(cze_h.at[pl.ds(base, _SW)], cz_v)

        def body(i, _):
            sl = pl.ds(i * 16, 16)
            iv = idx_v[sl]
            nx_v[sl] = plsc.load_gather(x_v, [iv]) - cx_v[sl]
            ny_v[sl] = plsc.load_gather(y_v, [iv]) - cy_v[sl]
            nz_v[sl] = plsc.load_gather(z_v, [iv]) - cz_v[sl]
            return 0

        lax.fori_loop(0, _SW // 16, body, 0)
        pltpu.sync_copy(nx_v, nx_h.at[pl.ds(base, _SW)])
        pltpu.sync_copy(ny_v, ny_h.at[pl.ds(base, _SW)])
        pltpu.sync_copy(nz_v, nz_h.at[pl.ds(base, _SW)])

    return k(x, y, z, idx_flat, cxe, cye, cze)


def kernel(xyz):
    xt = xyz.transpose(0, 2, 1)
    x, y, z = xt[:, 0, :], xt[:, 1, :], xt[:, 2, :]
    cx, cy, cz = _fps_centers(x, y, z)
    center = jnp.stack([cx, cy, cz], axis=-1)
    knn_idx = _knn_indices(x, y, z, cx, cy, cz)
    cxe = jnp.repeat(cx.reshape(-1), _K)
    cye = jnp.repeat(cy.reshape(-1), _K)
    cze = jnp.repeat(cz.reshape(-1), _K)
    nx, ny, nz = _sc_gather(
        x.reshape(-1), y.reshape(-1), z.reshape(-1),
        knn_idx.reshape(-1), cxe, cye, cze,
    )
    neighborhood = jnp.stack([nx, ny, nz], axis=-1).reshape(_B, _G, _K, 3)
    return neighborhood, center


# speedup vs baseline: 1.1353x; 1.1353x over previous
import functools

import jax
import jax.numpy as jnp
from jax import lax
from jax.experimental import pallas as pl
from jax.experimental.pallas import tpu as pltpu
from jax.experimental.pallas import tpu_sc as plsc

_B, _N, _G, _K = 8, 8192, 512, 32
_GB = 128
_NW = 32
_RW = (_B * _G) // _NW
_SW = _RW * _K


def _fps_body(x_ref, y_ref, z_ref, cx_ref, cy_ref, cz_ref):
    x = x_ref[...]
    y = y_ref[...]
    z = z_ref[...]
    nb = x.shape[0]
    col = jax.lax.broadcasted_iota(jnp.int32, (nb, _N), 1)
    gcol = jax.lax.broadcasted_iota(jnp.int32, (nb, _G), 1)

    def step(t, carry):
        dist, far, cxs, cys, czs = carry
        sel = col == far
        cx = jnp.sum(jnp.where(sel, x, 0.0), axis=1, keepdims=True)
        cy = jnp.sum(jnp.where(sel, y, 0.0), axis=1, keepdims=True)
        cz = jnp.sum(jnp.where(sel, z, 0.0), axis=1, keepdims=True)
        gsel = gcol == t
        cxs = jnp.where(gsel, cx, cxs)
        cys = jnp.where(gsel, cy, cys)
        czs = jnp.where(gsel, cz, czs)
        dx = x - cx
        dy = y - cy
        dz = z - cz
        d = (dx * dx + dy * dy) + dz * dz
        dist = jnp.minimum(dist, d)
        m = jnp.max(dist, axis=1, keepdims=True)
        far = jnp.min(jnp.where(dist == m, col, _N), axis=1, keepdims=True)
        return dist, far, cxs, cys, czs

    dist0 = jnp.full((nb, _N), 1e10, jnp.float32)
    far0 = jnp.zeros((nb, 1), jnp.int32)
    c0 = jnp.zeros((nb, _G), jnp.float32)
    _, _, cxs, cys, czs = jax.lax.fori_loop(
        0, _G, step, (dist0, far0, c0, c0, c0)
    )
    cx_ref[...] = cxs
    cy_ref[...] = cys
    cz_ref[...] = czs


def _rne_bf16(v):
    u = jax.lax.bitcast_convert_type(v, jnp.uint32)
    r = (u + jnp.uint32(0x7FFF) + ((u >> 16) & jnp.uint32(1))) & jnp.uint32(
        0xFFFF0000
    )
    return jax.lax.bitcast_convert_type(r, jnp.float32)


def _twosum(a, b):
    s = a + b
    bb = s - a
    err = (a - (s - bb)) + (b - bb)
    return s, err


def _knn_body(x_ref, y_ref, z_ref, cx_ref, cy_ref, cz_ref, out_ref):
    x = x_ref[0]
    y = y_ref[0]
    z = z_ref[0]
    cx = cx_ref[0]
    cy = cy_ref[0]
    cz = cz_ref[0]
    p2 = (x * x + y * y) + z * z
    c2 = (cx * cx + cy * cy) + cz * cz
    xb, yb, zb = _rne_bf16(x), _rne_bf16(y), _rne_bf16(z)
    cxb, cyb, czb = _rne_bf16(cx), _rne_bf16(cy), _rne_bf16(cz)
    p0 = cxb * xb
    p1 = cyb * yb
    p2b = czb * zb
    s1, e1 = _twosum(p0, p1)
    s2, e2 = _twosum(s1, p2b)
    dot = s2 + (e1 + e2)
    d2 = (c2 + p2) - 2.0 * dot
    col = jax.lax.broadcasted_iota(jnp.int32, (_GB, _N), 1)
    jcol = jax.lax.broadcasted_iota(jnp.int32, (_GB, _K), 1)
    inds = jnp.zeros((_GB, _K), jnp.int32)
    for j in range(_K):
        m = jnp.min(d2, axis=1, keepdims=True)
        idxc = jnp.where(d2 == m, col, _N)
        idx = jnp.min(idxc, axis=1, keepdims=True)
        d2 = jnp.where(idxc == idx, jnp.float32(jnp.inf), d2)
        inds = jnp.where(jcol == j, idx, inds)
    out_ref[...] = inds[None]


def _fps_centers(x, y, z):
    full = pl.BlockSpec((_B, _N), lambda: (0, 0))
    outs = jax.ShapeDtypeStruct((_B, _G), jnp.float32)
    return pl.pallas_call(
        _fps_body,
        grid=(),
        in_specs=[full, full, full],
        out_specs=[pl.BlockSpec((_B, _G), lambda: (0, 0))] * 3,
        out_shape=[outs, outs, outs],
    )(x, y, z)


def _knn_indices(x, y, z, cx, cy, cz):
    row = pl.BlockSpec((1, 1, _N), lambda b, g: (b, 0, 0))
    cen = pl.BlockSpec((1, _GB, 1), lambda b, g: (b, g, 0))
    return pl.pallas_call(
        _knn_body,
        grid=(_B, _G // _GB),
        in_specs=[row, row, row, cen, cen, cen],
        out_specs=pl.BlockSpec((1, _GB, _K), lambda b, g: (b, g, 0)),
        out_shape=jax.ShapeDtypeStruct((_B, _G, _K), jnp.int32),
        compiler_params=pltpu.CompilerParams(
            dimension_semantics=("parallel", "parallel")
        ),
    )(
        x.reshape(_B, 1, _N),
        y.reshape(_B, 1, _N),
        z.reshape(_B, 1, _N),
        cx.reshape(_B, _G, 1),
        cy.reshape(_B, _G, 1),
        cz.reshape(_B, _G, 1),
    )


def _sc_gather(x, y, z, idx_flat, cxe, cye, cze):
    mesh = plsc.VectorSubcoreMesh(core_axis_name="c", subcore_axis_name="s")
    out_t = jax.ShapeDtypeStruct((_B * _G * _K,), jnp.float32)

    @functools.partial(
        pl.kernel,
        mesh=mesh,
        out_type=[out_t, out_t, out_t],
        compiler_params=pltpu.CompilerParams(needs_layout_passes=False),
        scratch_types=[
            pltpu.VMEM((_SW,), jnp.int32),
            pltpu.VMEM((_N,), jnp.float32),
            pltpu.VMEM((_N,), jnp.float32),
            pltpu.VMEM((_N,), jnp.float32),
            pltpu.VMEM((_SW,), jnp.float32),
            pltpu.VMEM((_SW,), jnp.float32),
            pltpu.VMEM((_SW,), jnp.float32),
            pltpu.VMEM((_SW,), jnp.float32),
            pltpu.VMEM((_SW,), jnp.float32),
            pltpu.VMEM((_SW,), jnp.float32),
        ],
    )
    def k(x_h, y_h, z_h, idx_h, cxe_h, cye_h, cze_h, nx_h, ny_h, nz_h,
          idx_v, x_v, y_v, z_v, cx_v, cy_v, cz_v, nx_v, ny_v, nz_v):
        wid = lax.axis_index("s") * 2 + lax.axis_index("c")
        base = wid * _SW
        b = wid // (_NW // _B)
        pltpu.sync_copy(x_h.at[pl.ds(b * _N, _N)], x_v)
        pltpu.sync_copy(y_h.at[pl.ds(b * _N, _N)], y_v)
        pltpu.sync_copy(z_h.at[pl.ds(b * _N, _N)], z_v)
        pltpu.sync_copy(idx_h.at[pl.ds(base, _SW)], idx_v)
        pltpu.sync_copy(cxe_h.at[pl.ds(base, _SW)], cx_v)
        pltpu.sync_copy(cye_h.at[pl.ds(base, _SW)], cy_v)
        pltpu.sync_copy(cze_h.at[pl.ds(base, _SW)], cz_v)

        def body(i, _):
            sl = pl.ds(i * 16, 16)
            iv = idx_v[sl]
            nx_v[sl] = plsc.load_gather(x_v, [iv]) - cx_v[sl]
            ny_v[sl] = plsc.load_gather(y_v, [iv]) - cy_v[sl]
            nz_v[sl] = plsc.load_gather(z_v, [iv]) - cz_v[sl]
            return 0

        lax.fori_loop(0, _SW // 16, body, 0)
        pltpu.sync_copy(nx_v, nx_h.at[pl.ds(base, _SW)])
        pltpu.sync_copy(ny_v, ny_h.at[pl.ds(base, _SW)])
        pltpu.sync_copy(nz_v, nz_h.at[pl.ds(base, _SW)])

    return k(x, y, z, idx_flat, cxe, cye, cze)


def kernel(xyz):
    xt = xyz.transpose(0, 2, 1)
    x, y, z = xt[:, 0, :], xt[:, 1, :], xt[:, 2, :]
    cx, cy, cz = _fps_centers(x, y, z)
    center = jnp.stack([cx, cy, cz], axis=-1)
    knn_idx = _knn_indices(x, y, z, cx, cy, cz)
    cxe = jnp.repeat(cx.reshape(-1), _K)
    cye = jnp.repeat(cy.reshape(-1), _K)
    cze = jnp.repeat(cz.reshape(-1), _K)
    nx, ny, nz = _sc_gather(
        x.reshape(-1), y.reshape(-1), z.reshape(-1),
        knn_idx.reshape(-1), cxe, cye, cze,
    )
    neighborhood = jnp.stack([nx, ny, nz], axis=-1).reshape(_B, _G, _K, 3)
    return neighborhood, center
